# baseline (speedup 1.0000x reference)
import jax
import jax.numpy as jnp
from jax import lax
from jax.experimental import pallas as pl
from jax.experimental.pallas import tpu as pltpu
from jax.experimental.pallas import tpu_sc as plsc

BATCH = 4096
NUM_POS = 77
EMBED_DIM = 768
LANES = 16
NUM_CORES = 2
NUM_SUBCORES = 16
NUM_WORKERS = NUM_CORES * NUM_SUBCORES
ROWS = BATCH * NUM_POS
ROWS_PER_WORKER = ROWS // NUM_WORKERS
CHUNK = 16
NCHUNK = ROWS_PER_WORKER // CHUNK
NBUF = 4
NGROUP = NCHUNK // NBUF
COLS = EMBED_DIM // LANES


def _body(idx_hbm, table_hbm, pos_hbm, out_hbm, idx_v, pos_v,
          buf0, buf1, buf2, buf3, g0, g1, g2, g3, w0, w1, w2, w3):
    bufs = (buf0, buf1, buf2, buf3)
    gsem = (g0, g1, g2, g3)
    wsem = (w0, w1, w2, w3)
    wid = lax.axis_index("s") * NUM_CORES + lax.axis_index("c")
    base = wid * ROWS_PER_WORKER

    pltpu.sync_copy(idx_hbm.at[pl.ds(base, ROWS_PER_WORKER)], idx_v)
    pltpu.sync_copy(pos_hbm, pos_v)

    def start_gather(ch, b):
        pltpu.async_copy(
            table_hbm.at[idx_v.at[pl.ds(ch * CHUNK, CHUNK)]], bufs[b], gsem[b])

    def wait_gather(b):
        pltpu.make_async_copy(table_hbm.at[idx_v.at[pl.ds(0, CHUNK)]],
                              bufs[b], gsem[b]).wait()

    def start_write(ch, b):
        pltpu.async_copy(bufs[b], out_hbm.at[pl.ds(base + ch * CHUNK, CHUNK)],
                         wsem[b])

    def wait_write(b):
        pltpu.make_async_copy(bufs[b], out_hbm.at[pl.ds(base, CHUNK)],
                              wsem[b]).wait()

    def add_pos(i, b):
        p0 = lax.rem(i * CHUNK, NUM_POS)

        def row_step(j, _):
            p = p0 + j
            p = lax.select(p >= NUM_POS, p - NUM_POS, p)
            for c in range(COLS):
                sl = pl.ds(c * LANES, LANES)
                bufs[b][j, sl] = bufs[b][j, sl] + pos_v[p, sl]
            return 0

        lax.fori_loop(0, CHUNK, row_step, 0, unroll=False)

    start_gather(0, 0)
    start_gather(1, 1)

    def group(g, _):
        for b in range(NBUF):
            i = g * NBUF + b
            f = i + 2
            fb = (b + 2) % NBUF
            @pl.when(f >= NBUF)
            def _():
                wait_write(fb)
            start_gather(f, fb)
            wait_gather(b)
            add_pos(i, b)
            start_write(i, b)
        return 0

    lax.fori_loop(0, NGROUP - 1, group, 0, unroll=False)

    for b in range(NBUF):
        i = (NGROUP - 1) * NBUF + b
        f = i + 2
        if f < NCHUNK:
            fb = (b + 2) % NBUF
            wait_write(fb)
            start_gather(f, fb)
        wait_gather(b)
        add_pos(jnp.int32(i), b)
        start_write(i, b)

    for b in range(NBUF):
        wait_write(b)


@jax.jit
def _embed(idx, token_table, pos_table):
    mesh = plsc.VectorSubcoreMesh(core_axis_name="c", subcore_axis_name="s")
    fn = pl.kernel(
        _body,
        out_type=jax.ShapeDtypeStruct((ROWS, EMBED_DIM), jnp.float32),
        mesh=mesh,
        compiler_params=pltpu.CompilerParams(use_tc_tiling_on_sc=False),
        scratch_types=[
            pltpu.VMEM((ROWS_PER_WORKER,), jnp.int32),
            pltpu.VMEM((NUM_POS, EMBED_DIM), jnp.float32),
        ] + [pltpu.VMEM((CHUNK, EMBED_DIM), jnp.float32)] * NBUF
          + [pltpu.SemaphoreType.DMA] * (2 * NBUF),
    )
    return fn(idx, token_table, pos_table)


def kernel(input_tokens, token_table, pos_table):
    idx = input_tokens.astype(jnp.int32).reshape(-1)
    out = _embed(idx, token_table, pos_table)
    return out.reshape(BATCH, NUM_POS, EMBED_DIM)


# speedup vs baseline: 1.1786x; 1.1786x over previous
import jax
import jax.numpy as jnp
from jax import lax
from jax.experimental import pallas as pl
from jax.experimental.pallas import tpu as pltpu
from jax.experimental.pallas import tpu_sc as plsc

BATCH = 4096
NUM_POS = 77
EMBED_DIM = 768
LANES = 16
NUM_CORES = 2
NUM_SUBCORES = 16
NUM_WORKERS = NUM_CORES * NUM_SUBCORES
ROWS = BATCH * NUM_POS
ROWS_PER_WORKER = ROWS // NUM_WORKERS
CHUNK = 16
NCHUNK = ROWS_PER_WORKER // CHUNK
NBUF = 4
NGROUP = NCHUNK // NBUF
COLS = EMBED_DIM // LANES


def _body(idx_hbm, table_hbm, pos_hbm, out_hbm, idx_v, pos_v,
          buf0, buf1, buf2, buf3, g0, g1, g2, g3, w0, w1, w2, w3):
    bufs = (buf0, buf1, buf2, buf3)
    gsem = (g0, g1, g2, g3)
    wsem = (w0, w1, w2, w3)
    wid = lax.axis_index("s") * NUM_CORES + lax.axis_index("c")
    base = wid * ROWS_PER_WORKER

    pltpu.sync_copy(idx_hbm.at[pl.ds(base, ROWS_PER_WORKER)], idx_v)
    pltpu.sync_copy(pos_hbm, pos_v)

    def start_gather(ch, b):
        pltpu.async_copy(
            table_hbm.at[idx_v.at[pl.ds(ch * CHUNK, CHUNK)]], bufs[b], gsem[b])

    def wait_gather(b):
        pltpu.make_async_copy(table_hbm.at[idx_v.at[pl.ds(0, CHUNK)]],
                              bufs[b], gsem[b]).wait()

    def start_write(ch, b):
        pltpu.async_copy(bufs[b], out_hbm.at[pl.ds(base + ch * CHUNK, CHUNK)],
                         wsem[b])

    def wait_write(b):
        pltpu.make_async_copy(bufs[b], out_hbm.at[pl.ds(base, CHUNK)],
                              wsem[b]).wait()

    def add_pos(i, b):
        p0 = lax.rem(i * CHUNK, NUM_POS)

        def row_step(j, _):
            p = p0 + j
            p = lax.select(p >= NUM_POS, p - NUM_POS, p)
            for c in range(COLS):
                sl = pl.ds(c * LANES, LANES)
                plsc.addupdate(bufs[b].at[j, sl], pos_v[p, sl])
            return 0

        lax.fori_loop(0, CHUNK, row_step, 0, unroll=False)

    start_gather(0, 0)
    start_gather(1, 1)

    def group(g, _):
        for b in range(NBUF):
            i = g * NBUF + b
            f = i + 2
            fb = (b + 2) % NBUF
            @pl.when(f >= NBUF)
            def _():
                wait_write(fb)
            start_gather(f, fb)
            wait_gather(b)
            add_pos(i, b)
            start_write(i, b)
        return 0

    lax.fori_loop(0, NGROUP - 1, group, 0, unroll=False)

    for b in range(NBUF):
        i = (NGROUP - 1) * NBUF + b
        f = i + 2
        if f < NCHUNK:
            fb = (b + 2) % NBUF
            wait_write(fb)
            start_gather(f, fb)
        wait_gather(b)
        add_pos(jnp.int32(i), b)
        start_write(i, b)

    for b in range(NBUF):
        wait_write(b)


@jax.jit
def _embed(idx, token_table, pos_table):
    mesh = plsc.VectorSubcoreMesh(core_axis_name="c", subcore_axis_name="s")
    fn = pl.kernel(
        _body,
        out_type=jax.ShapeDtypeStruct((ROWS, EMBED_DIM), jnp.float32),
        mesh=mesh,
        compiler_params=pltpu.CompilerParams(use_tc_tiling_on_sc=False),
        scratch_types=[
            pltpu.VMEM((ROWS_PER_WORKER,), jnp.int32),
            pltpu.VMEM((NUM_POS, EMBED_DIM), jnp.float32),
        ] + [pltpu.VMEM((CHUNK, EMBED_DIM), jnp.float32)] * NBUF
          + [pltpu.SemaphoreType.DMA] * (2 * NBUF),
    )
    return fn(idx, token_table, pos_table)


def kernel(input_tokens, token_table, pos_table):
    idx = input_tokens.astype(jnp.int32).reshape(-1)
    out = _embed(idx, token_table, pos_table)
    return out.reshape(BATCH, NUM_POS, EMBED_DIM)


# speedup vs baseline: 1.5159x; 1.2863x over previous
import jax
import jax.numpy as jnp
from jax import lax
from jax.experimental import pallas as pl
from jax.experimental.pallas import tpu as pltpu
from jax.experimental.pallas import tpu_sc as plsc

BATCH = 4096
NUM_POS = 77
EMBED_DIM = 768
LANES = 16
PAIRS = EMBED_DIM // 32
NUM_CORES = 2
NUM_WORKERS = 32
BE_PER_WORKER = BATCH // NUM_WORKERS
IDX_BLK = 8
WAVES = (16, 16, 16, 16, 13)


def _body(idx_hbm, table_hbm, pos_hbm, out_hbm, idx_v, pos_v, buf2, wv0, wv1,
          isem, g0, g1, wsem):
    waves = (wv0, wv1)
    gsem = (g0, g1)
    wid = lax.axis_index("s") * NUM_CORES + lax.axis_index("c")
    base = wid * BE_PER_WORKER

    pltpu.sync_copy(pos_hbm, pos_v)

    def fire_wave(i, w):
        w0 = 16 * w
        n = WAVES[w]
        off = min(w0, NUM_POS - LANES)
        v = idx_v[i, pl.ds(off, LANES)]
        for m in range(w0 - off, w0 - off + n):
            r = m - (w0 - off)
            t = v[m]
            pltpu.async_copy(
                table_hbm.at[pl.ds(t * EMBED_DIM, EMBED_DIM)],
                waves[w % 2].at[pl.ds(r * EMBED_DIM, EMBED_DIM)],
                gsem[w % 2])

    def drain_assemble_wave(w):
        w0 = 16 * w
        n = WAVES[w]
        wb = waves[w % 2]
        sem = gsem[w % 2]

        def row_step(jj, _):
            pltpu.make_async_copy(
                table_hbm.at[pl.ds(0, EMBED_DIM)],
                wb.at[pl.ds(jj * EMBED_DIM, EMBED_DIM)], sem).wait()
            j = w0 + jj
            for c in range(PAIRS):
                q = plsc.bitcast(
                    pos_v[pl.ds(j * (EMBED_DIM // 2) + c * LANES, LANES)],
                    jnp.int32)
                lo = plsc.bitcast(q << 16, jnp.float32)
                hi = plsc.bitcast(q & jnp.int32(-65536), jnp.float32)
                ta = wb[pl.ds(jj * EMBED_DIM + c * 32, LANES)]
                tb = wb[pl.ds(jj * EMBED_DIM + c * 32 + LANES, LANES)]
                buf2[j, pl.ds(c * 32, LANES)] = ta + lo
                buf2[j, pl.ds(c * 32 + LANES, LANES)] = tb + hi
            return 0

        lax.fori_loop(0, n, row_step, 0, unroll=False)

    def blk_step(k, _):
        pltpu.async_copy(idx_hbm.at[pl.ds(base + k * IDX_BLK, IDX_BLK)],
                         idx_v, isem).wait()

        def elem_step(i, _):
            e = k * IDX_BLK + i
            fire_wave(i, 0)
            fire_wave(i, 1)
            @pl.when(jnp.logical_or(k > 0, i > 0))
            def _():
                pltpu.make_async_copy(buf2, out_hbm.at[base], wsem).wait()
            for w in range(len(WAVES)):
                drain_assemble_wave(w)
                if w + 2 < len(WAVES):
                    fire_wave(i, w + 2)
            pltpu.async_copy(buf2, out_hbm.at[base + e], wsem)
            return 0

        lax.fori_loop(0, IDX_BLK, elem_step, 0, unroll=False)
        return 0

    lax.fori_loop(0, BE_PER_WORKER // IDX_BLK, blk_step, 0, unroll=False)
    pltpu.make_async_copy(buf2, out_hbm.at[base], wsem).wait()


@jax.jit
def _embed(idx, table_flat, pos_sh):
    mesh = plsc.VectorSubcoreMesh(core_axis_name="c", subcore_axis_name="s")
    fn = pl.kernel(
        _body,
        out_type=jax.ShapeDtypeStruct((BATCH, NUM_POS, EMBED_DIM), jnp.float32),
        mesh=mesh,
        compiler_params=pltpu.CompilerParams(use_tc_tiling_on_sc=True,
                                             needs_layout_passes=False),
        scratch_types=[
            pltpu.VMEM((IDX_BLK, NUM_POS), jnp.int32),
            pltpu.VMEM((NUM_POS * EMBED_DIM,), jnp.bfloat16),
            pltpu.VMEM((NUM_POS, EMBED_DIM), jnp.float32),
            pltpu.VMEM((LANES * EMBED_DIM,), jnp.float32),
            pltpu.VMEM((LANES * EMBED_DIM,), jnp.float32),
            pltpu.SemaphoreType.DMA,
            pltpu.SemaphoreType.DMA,
            pltpu.SemaphoreType.DMA,
            pltpu.SemaphoreType.DMA,
        ],
    )
    return fn(idx, table_flat, pos_sh)


def kernel(input_tokens, token_table, pos_table):
    idx = input_tokens.astype(jnp.int32)
    table_flat = token_table.reshape(-1)
    pos_sh = (pos_table.reshape(NUM_POS, PAIRS, 2, LANES)
              .transpose(0, 1, 3, 2)
              .reshape(NUM_POS * EMBED_DIM)
              .astype(jnp.bfloat16))
    pos_pairs = lax.bitcast_convert_type(
        pos_sh.reshape(NUM_POS * EMBED_DIM // 2, 2), jnp.float32)
    return _embed(idx, table_flat, pos_pairs)
